# baseline (speedup 1.0000x reference)
import functools

import jax
import jax.numpy as jnp
from jax.experimental import pallas as pl
from jax.experimental.pallas import tpu as pltpu

EPS = 1e-5
QEPS = 1e-8
L_BLK = 512


def _prep_body(qin_ref, wq_ref, wk_ref, wv_ref, wo_ref,
               kq_ref, wvq_ref, woq_ref, *, scale):
    qin = qin_ref[...]
    g = jnp.clip(jnp.max(jnp.abs(qin), axis=-1, keepdims=True), QEPS, None)
    qa = jnp.round(qin * (127.0 / g)) * (g / 127.0)
    wq = wq_ref[...]
    sq = jnp.mean(jnp.abs(wq)) + QEPS
    wqq = jnp.clip(jnp.round(wq / sq), -1.0, 1.0) * sq
    q_flat = jax.lax.dot_general(qa, wqq, (((1,), (1,)), ((), ())),
                                 preferred_element_type=jnp.float32)
    wk = wk_ref[...]
    sk = jnp.mean(jnp.abs(wk)) + QEPS
    wkq = jnp.clip(jnp.round(wk / sk), -1.0, 1.0) * sk
    kq = jax.lax.dot_general(q_flat, wkq, (((1,), (0,)), ((), ())),
                             preferred_element_type=jnp.float32)
    kq_ref[...] = kq * scale
    wv = wv_ref[...]
    sv = jnp.mean(jnp.abs(wv)) + QEPS
    wvq_ref[...] = jnp.clip(jnp.round(wv / sv), -1.0, 1.0) * sv
    wo = wo_ref[...]
    so = jnp.mean(jnp.abs(wo)) + QEPS
    woq_ref[...] = jnp.clip(jnp.round(wo / so), -1.0, 1.0) * so


def _mega_body(kq_ref, nw_ref, wvq_ref, woq_ref, rp_ref, ra_ref, o_ref,
               acc_ref, m_ref, s_ref, corr_ref, *, d_model, nl, n_b):
    b = pl.program_id(0)
    l = pl.program_id(1)

    @pl.when(b > 0)
    def _add():
        o_ref[...] = ra_ref[...] + corr_ref[...]

    @pl.when(b < n_b)
    def _pool():
        @pl.when(l == 0)
        def _init():
            m_ref[...] = jnp.full_like(m_ref, -1e30)
            s_ref[...] = jnp.zeros_like(s_ref)
            acc_ref[...] = jnp.zeros_like(acc_ref)

        r = rp_ref[0]
        w = nw_ref[...]
        ssq = jnp.sum(r * r, axis=-1, keepdims=True)
        rs = jax.lax.rsqrt(ssq / d_model + EPS)
        u = r * w
        gu = jnp.max(jnp.abs(u), axis=-1, keepdims=True)
        g = jnp.clip(rs * gu, QEPS, None)
        rq = jnp.round(u * (rs * (127.0 / g)))
        c2 = g * (1.0 / 127.0)
        lg = jnp.sum(rq * kq_ref[...], axis=-1, keepdims=True) * c2
        rq_bf = rq.astype(jnp.bfloat16)

        m_old = m_ref[...]
        m_new = jnp.maximum(m_old, jnp.max(lg, axis=0, keepdims=True))
        alpha = jnp.exp(m_old - m_new)
        pr = jnp.exp(lg - m_new)
        s_ref[...] = s_ref[...] * alpha + jnp.sum(pr, axis=0, keepdims=True)
        pw = (pr * c2).astype(jnp.bfloat16)
        acc_ref[...] = acc_ref[...] * alpha + jax.lax.dot_general(
            pw, rq_bf, (((0,), (0,)), ((), ())),
            preferred_element_type=jnp.float32)
        m_ref[...] = m_new

        @pl.when(l == nl - 1)
        def _tail():
            sx = acc_ref[...] / s_ref[...]
            summary = jax.lax.dot_general(
                sx, wvq_ref[...], (((1,), (1,)), ((), ())),
                preferred_element_type=jnp.float32)
            gs = jnp.clip(jnp.max(jnp.abs(summary), axis=-1, keepdims=True),
                          QEPS, None)
            qs = jnp.round(summary * (127.0 / gs)) * (gs / 127.0)
            corr_ref[...] = jax.lax.dot_general(
                qs, woq_ref[...], (((1,), (1,)), ((), ())),
                preferred_element_type=jnp.float32)


def kernel(meta_real, meta_imag, residual, wq_w, wk_w, wv_w, wo_w, norm_w):
    B, L, D = residual.shape
    scale = D ** (-0.5)
    q_input = jnp.stack([meta_real, meta_imag], axis=-1).reshape(1, -1)
    nw = norm_w.reshape(1, D)

    kq, wvq, woq = pl.pallas_call(
        functools.partial(_prep_body, scale=scale),
        out_shape=(
            jax.ShapeDtypeStruct((1, D), jnp.float32),
            jax.ShapeDtypeStruct((D, D), jnp.float32),
            jax.ShapeDtypeStruct((D, D), jnp.float32),
        ),
    )(q_input, wq_w, wk_w, wv_w, wo_w)

    nl = L // L_BLK
    nb = B
    out = pl.pallas_call(
        functools.partial(_mega_body, d_model=D, nl=nl, n_b=nb),
        grid=(B + 1, nl),
        in_specs=[
            pl.BlockSpec((1, D), lambda b, l: (0, 0)),
            pl.BlockSpec((1, D), lambda b, l: (0, 0)),
            pl.BlockSpec((D, D), lambda b, l: (0, 0)),
            pl.BlockSpec((D, D), lambda b, l: (0, 0)),
            pl.BlockSpec((1, L_BLK, D),
                         lambda b, l: (jnp.minimum(b, nb - 1),
                                       jnp.where(b < nb, l, 0), 0)),
            pl.BlockSpec((1, L_BLK, D),
                         lambda b, l: (jnp.maximum(b - 1, 0),
                                       jnp.where(b > 0, l, 0), 0)),
        ],
        out_specs=pl.BlockSpec((1, L_BLK, D),
                               lambda b, l: (jnp.maximum(b - 1, 0),
                                             jnp.where(b > 0, l, 0), 0)),
        out_shape=jax.ShapeDtypeStruct((B, L, D), jnp.float32),
        scratch_shapes=[
            pltpu.VMEM((1, D), jnp.float32),
            pltpu.VMEM((1, 1), jnp.float32),
            pltpu.VMEM((1, 1), jnp.float32),
            pltpu.VMEM((1, D), jnp.float32),
        ],
        compiler_params=pltpu.CompilerParams(
            dimension_semantics=("arbitrary", "arbitrary"),
            vmem_limit_bytes=56 * 1024 * 1024),
    )(kq, nw, wvq, woq, residual, residual)
    return out


# speedup vs baseline: 1.1510x; 1.1510x over previous
import functools

import jax
import jax.numpy as jnp
from jax.experimental import pallas as pl
from jax.experimental.pallas import tpu as pltpu

EPS = 1e-5
QEPS = 1e-8
L_BLK = 1024


def _prep_body(qin_ref, wq_ref, wk_ref, wv_ref, wo_ref,
               kqm_ref, wvq_ref, woq_ref, *, scale):
    qin = qin_ref[...]
    g = jnp.clip(jnp.max(jnp.abs(qin), axis=-1, keepdims=True), QEPS, None)
    qa = jnp.round(qin * (127.0 / g)) * (g / 127.0)
    wq = wq_ref[...]
    sq = jnp.mean(jnp.abs(wq)) + QEPS
    wqq = jnp.clip(jnp.round(wq / sq), -1.0, 1.0) * sq
    q_flat = jax.lax.dot_general(qa, wqq, (((1,), (1,)), ((), ())),
                                 preferred_element_type=jnp.float32)
    wk = wk_ref[...]
    sk = jnp.mean(jnp.abs(wk)) + QEPS
    wkq = jnp.clip(jnp.round(wk / sk), -1.0, 1.0) * sk
    kq = jax.lax.dot_general(q_flat, wkq, (((1,), (0,)), ((), ())),
                             preferred_element_type=jnp.float32)
    kqm_ref[...] = jnp.broadcast_to((kq * scale).astype(jnp.bfloat16),
                                    kqm_ref.shape)
    wv = wv_ref[...]
    sv = jnp.mean(jnp.abs(wv)) + QEPS
    wvq_ref[...] = jnp.clip(jnp.round(wv / sv), -1.0, 1.0) * sv
    wo = wo_ref[...]
    so = jnp.mean(jnp.abs(wo)) + QEPS
    woq_ref[...] = jnp.clip(jnp.round(wo / so), -1.0, 1.0) * so


def _mega_body(kqm_ref, nw_ref, wvq_ref, woq_ref, r_ref, o_ref,
               resbuf_ref, acc_ref, m_ref, s_ref, corr_ref, *,
               d_model, nl):
    p = pl.program_id(1)
    l = pl.program_id(2)
    off = pl.multiple_of(l * L_BLK, L_BLK)

    @pl.when(p == 0)
    def _pool():
        @pl.when(l == 0)
        def _init():
            m_ref[...] = jnp.full_like(m_ref, -1e30)
            s_ref[...] = jnp.zeros_like(s_ref)
            acc_ref[...] = jnp.zeros_like(acc_ref)

        r = r_ref[0]
        resbuf_ref[pl.ds(off, L_BLK), :] = r.astype(jnp.bfloat16)
        w = nw_ref[...]
        ssq = jnp.sum(r * r, axis=-1, keepdims=True)
        rs = jax.lax.rsqrt(ssq / d_model + EPS)
        u = r * w
        gu = jnp.max(jnp.abs(u), axis=-1, keepdims=True)
        g = jnp.clip(rs * gu, QEPS, None)
        rq = jnp.round(u * (rs * (127.0 / g))).astype(jnp.bfloat16)
        c2 = g * (1.0 / 127.0)
        lg = jax.lax.dot_general(
            rq, kqm_ref[...], (((1,), (1,)), ((), ())),
            preferred_element_type=jnp.float32)[:, :1] * c2

        m_old = m_ref[...]
        m_new = jnp.maximum(m_old, jnp.max(lg, axis=0, keepdims=True))
        alpha = jnp.exp(m_old - m_new)
        pr = jnp.exp(lg - m_new)
        s_ref[...] = s_ref[...] * alpha + jnp.sum(pr, axis=0, keepdims=True)
        pw = (pr * c2).astype(jnp.bfloat16)
        acc_ref[...] = acc_ref[...] * alpha + jax.lax.dot_general(
            pw, rq, (((0,), (0,)), ((), ())),
            preferred_element_type=jnp.float32)
        m_ref[...] = m_new

        @pl.when(l == nl - 1)
        def _tail():
            sx = acc_ref[...] / s_ref[...]
            summary = jax.lax.dot_general(
                sx, wvq_ref[...], (((1,), (1,)), ((), ())),
                preferred_element_type=jnp.float32)
            gs = jnp.clip(jnp.max(jnp.abs(summary), axis=-1, keepdims=True),
                          QEPS, None)
            qs = jnp.round(summary * (127.0 / gs)) * (gs / 127.0)
            corr_ref[...] = jax.lax.dot_general(
                qs, woq_ref[...], (((1,), (1,)), ((), ())),
                preferred_element_type=jnp.float32)

    @pl.when(p == 1)
    def _add():
        o_ref[...] = (resbuf_ref[pl.ds(off, L_BLK), :].astype(jnp.float32)
                      + corr_ref[...])[None]


def kernel(meta_real, meta_imag, residual, wq_w, wk_w, wv_w, wo_w, norm_w):
    B, L, D = residual.shape
    scale = D ** (-0.5)
    q_input = jnp.stack([meta_real, meta_imag], axis=-1).reshape(1, -1)
    nw = norm_w.reshape(1, D)

    kqm, wvq, woq = pl.pallas_call(
        functools.partial(_prep_body, scale=scale),
        out_shape=(
            jax.ShapeDtypeStruct((128, D), jnp.bfloat16),
            jax.ShapeDtypeStruct((D, D), jnp.float32),
            jax.ShapeDtypeStruct((D, D), jnp.float32),
        ),
    )(q_input, wq_w, wk_w, wv_w, wo_w)

    nl = L // L_BLK
    out = pl.pallas_call(
        functools.partial(_mega_body, d_model=D, nl=nl),
        grid=(B, 2, nl),
        in_specs=[
            pl.BlockSpec((128, D), lambda b, p, l: (0, 0)),
            pl.BlockSpec((1, D), lambda b, p, l: (0, 0)),
            pl.BlockSpec((D, D), lambda b, p, l: (0, 0)),
            pl.BlockSpec((D, D), lambda b, p, l: (0, 0)),
            pl.BlockSpec((1, L_BLK, D),
                         lambda b, p, l: (b, jnp.where(p == 0, l, 0), 0)),
        ],
        out_specs=pl.BlockSpec((1, L_BLK, D),
                               lambda b, p, l: (b, jnp.where(p == 0, 0, l), 0)),
        out_shape=jax.ShapeDtypeStruct((B, L, D), jnp.float32),
        scratch_shapes=[
            pltpu.VMEM((L, D), jnp.bfloat16),
            pltpu.VMEM((1, D), jnp.float32),
            pltpu.VMEM((1, 1), jnp.float32),
            pltpu.VMEM((1, 1), jnp.float32),
            pltpu.VMEM((1, D), jnp.float32),
        ],
        compiler_params=pltpu.CompilerParams(
            dimension_semantics=("parallel", "arbitrary", "arbitrary"),
            vmem_limit_bytes=56 * 1024 * 1024),
    )(kqm, nw, wvq, woq, residual)
    return out


# speedup vs baseline: 1.2147x; 1.0553x over previous
import functools

import jax
import jax.numpy as jnp
from jax.experimental import pallas as pl
from jax.experimental.pallas import tpu as pltpu

EPS = 1e-5
QEPS = 1e-8
L_BLK = 1024


def _prep_body(qin_ref, wq_ref, wk_ref, wv_ref, wo_ref,
               kqm_ref, wvq_ref, woq_ref, scl_ref, *, scale):
    qin = qin_ref[...]
    g = jnp.clip(jnp.max(jnp.abs(qin), axis=-1, keepdims=True), QEPS, None)
    qa = jnp.round(qin * (127.0 / g)) * (g / 127.0)
    wq = wq_ref[...]
    sq = jnp.mean(jnp.abs(wq)) + QEPS
    wqq = jnp.clip(jnp.round(wq / sq), -1.0, 1.0) * sq
    q_flat = jax.lax.dot_general(qa, wqq, (((1,), (1,)), ((), ())),
                                 preferred_element_type=jnp.float32)
    wk = wk_ref[...]
    sk = jnp.mean(jnp.abs(wk)) + QEPS
    wkq = jnp.clip(jnp.round(wk / sk), -1.0, 1.0) * sk
    kq = jax.lax.dot_general(q_flat, wkq, (((1,), (0,)), ((), ())),
                             preferred_element_type=jnp.float32)
    kqm_ref[...] = jnp.broadcast_to((kq * scale).astype(jnp.bfloat16),
                                    kqm_ref.shape)
    wv = wv_ref[...]
    sv = jnp.mean(jnp.abs(wv)) + QEPS
    wvq_ref[...] = jnp.clip(jnp.round(wv / sv), -1.0, 1.0).astype(jnp.bfloat16)
    wo = wo_ref[...]
    so = jnp.mean(jnp.abs(wo)) + QEPS
    woq_ref[...] = jnp.clip(jnp.round(wo / so), -1.0, 1.0).astype(jnp.bfloat16)
    scl_ref[...] = jnp.broadcast_to(
        jnp.stack([sv, so]).reshape(2, 1), scl_ref.shape)


def _mega_body(kqm_ref, nw_ref, wvq_ref, woq_ref, scl_ref, r_ref, o_ref,
               resbuf_ref, rqbuf_ref, hbuf_ref, corr_ref, *,
               d_model, nl):
    p = pl.program_id(1)
    l = pl.program_id(2)
    off = pl.multiple_of(l * L_BLK, L_BLK)

    @pl.when(p == 0)
    def _pool():
        r = r_ref[0]
        resbuf_ref[pl.ds(off, L_BLK), :] = r.astype(jnp.bfloat16)
        u = r * nw_ref[...]
        ssq = jnp.sum(r * r, axis=-1, keepdims=True)
        rs = jax.lax.rsqrt(ssq / d_model + EPS)
        gu = jnp.max(jnp.abs(u), axis=-1, keepdims=True)
        g = jnp.clip(rs * gu, QEPS, None)
        rq = jnp.round(u * (rs * (127.0 / g))).astype(jnp.bfloat16)
        rqbuf_ref[pl.ds(off, L_BLK), :d_model] = rq
        c2 = g * (1.0 / 127.0)
        rqbuf_ref[pl.ds(off, L_BLK), d_model:] = jnp.broadcast_to(
            1.0 / c2, (L_BLK, 128)).astype(jnp.bfloat16)
        lg = jax.lax.dot_general(
            rq, kqm_ref[...], (((1,), (1,)), ((), ())),
            preferred_element_type=jnp.float32)[:, :1] * c2
        hbuf_ref[pl.ds(off, L_BLK), :] = lg + jnp.log(c2)

    @pl.when(jnp.logical_and(p == 1, l == 0))
    def _finalize():
        hv = hbuf_ref[...]
        m = jnp.max(hv, axis=0, keepdims=True)
        pw = jnp.exp(hv - m).astype(jnp.bfloat16)
        sxe = jax.lax.dot_general(
            pw, rqbuf_ref[...], (((0,), (0,)), ((), ())),
            preferred_element_type=jnp.float32)
        sx = sxe[:, :d_model] / sxe[:, d_model:d_model + 1]
        sv = scl_ref[:1, :1]
        so = scl_ref[1:2, :1]
        y = jax.lax.dot_general(
            sx.astype(jnp.bfloat16), wvq_ref[...], (((1,), (1,)), ((), ())),
            preferred_element_type=jnp.float32)
        gy = jnp.max(jnp.abs(y), axis=-1, keepdims=True)
        gs = jnp.clip(sv * gy, QEPS, None)
        rqy = jnp.round(y * (sv * 127.0 / gs)).astype(jnp.bfloat16)
        corr_ref[...] = jax.lax.dot_general(
            rqy, woq_ref[...], (((1,), (1,)), ((), ())),
            preferred_element_type=jnp.float32) * (gs * so * (1.0 / 127.0))

    @pl.when(p == 1)
    def _add():
        o_ref[...] = (resbuf_ref[pl.ds(off, L_BLK), :].astype(jnp.float32)
                      + corr_ref[...])[None]


def kernel(meta_real, meta_imag, residual, wq_w, wk_w, wv_w, wo_w, norm_w):
    B, L, D = residual.shape
    scale = D ** (-0.5)
    q_input = jnp.stack([meta_real, meta_imag], axis=-1).reshape(1, -1)
    nw = norm_w.reshape(1, D)

    kqm, wvq, woq, scl = pl.pallas_call(
        functools.partial(_prep_body, scale=scale),
        out_shape=(
            jax.ShapeDtypeStruct((128, D), jnp.bfloat16),
            jax.ShapeDtypeStruct((D, D), jnp.bfloat16),
            jax.ShapeDtypeStruct((D, D), jnp.bfloat16),
            jax.ShapeDtypeStruct((2, 128), jnp.float32),
        ),
    )(q_input, wq_w, wk_w, wv_w, wo_w)

    nl = L // L_BLK
    out = pl.pallas_call(
        functools.partial(_mega_body, d_model=D, nl=nl),
        grid=(B, 2, nl),
        in_specs=[
            pl.BlockSpec((128, D), lambda b, p, l: (0, 0)),
            pl.BlockSpec((1, D), lambda b, p, l: (0, 0)),
            pl.BlockSpec((D, D), lambda b, p, l: (0, 0)),
            pl.BlockSpec((D, D), lambda b, p, l: (0, 0)),
            pl.BlockSpec((2, 128), lambda b, p, l: (0, 0)),
            pl.BlockSpec((1, L_BLK, D),
                         lambda b, p, l: (b, jnp.where(p == 0, l, 0), 0)),
        ],
        out_specs=pl.BlockSpec((1, L_BLK, D),
                               lambda b, p, l: (b, jnp.where(p == 0, 0, l), 0)),
        out_shape=jax.ShapeDtypeStruct((B, L, D), jnp.float32),
        scratch_shapes=[
            pltpu.VMEM((L, D), jnp.bfloat16),
            pltpu.VMEM((L, D + 128), jnp.bfloat16),
            pltpu.VMEM((L, 1), jnp.float32),
            pltpu.VMEM((1, D), jnp.float32),
        ],
        compiler_params=pltpu.CompilerParams(
            dimension_semantics=("parallel", "arbitrary", "arbitrary"),
            vmem_limit_bytes=63 * 1024 * 1024),
    )(kqm, nw, wvq, woq, scl, residual)
    return out
